# Initial kernel scaffold; baseline (speedup 1.0000x reference)
import jax
import jax.numpy as jnp
from jax.experimental import pallas as pl


def kernel(x, emb, W1, b1, W2, b2, W3, b3):
    raise NotImplementedError("write your pallas kernel here")


# speedup vs baseline: 1.7236x; 1.7236x over previous
import functools

import jax
import jax.numpy as jnp
from jax import lax
from jax.experimental import pallas as pl
from jax.experimental.pallas import tpu as pltpu
from jax.experimental.pallas import tpu_sc as plsc

VOCAB = 100000
EMB = 300
HID = 4096
B = 4096
L = 200

NC = 2
NS = 16
NW = NC * NS
BPW = B // NW
LA = 104
LB = L - LA

_OFFS = tuple(16 * c for c in range(17)) + (272, 284)
_NCH = len(_OFFS)


def _sc_pool_body(x_hbm, emb_hbm, out_hbm, *refs):
    idxb = refs[0:2]
    bufs = refs[2:4]
    out_v = refs[4]
    gsem = refs[5:7]
    isem = refs[7:9]

    wid = lax.axis_index("s") * NC + lax.axis_index("c")
    base = wid * BPW
    zero = jnp.zeros((16,), jnp.float32)
    scale = jnp.full((16,), 1.0 / L, jnp.float32)

    def fire_idx(i, slot):
        pltpu.async_copy(x_hbm.at[base + i], idxb[slot], isem[slot])

    def wait_idx(i, slot):
        pltpu.make_async_copy(x_hbm.at[base + i], idxb[slot], isem[slot]).wait()

    def gather_a(slot):
        return pltpu.make_async_copy(
            emb_hbm.at[idxb[slot].at[pl.ds(0, LA)]], bufs[0], gsem[0])

    def gather_b(slot):
        return pltpu.make_async_copy(
            emb_hbm.at[idxb[slot].at[pl.ds(LA, LB)]],
            bufs[1].at[pl.ds(0, LB)], gsem[1])

    def reduce_rows(buf, n):
        @pl.loop(0, n, init_carry=(zero,) * _NCH)
        def sums(r, carry):
            return tuple(carry[c] + buf[r, pl.ds(_OFFS[c], 16)]
                         for c in range(_NCH))
        return sums

    fire_idx(0, 0)
    fire_idx(1, 1)
    wait_idx(0, 0)
    gather_a(0).start()

    @pl.loop(0, BPW // 2)
    def _(t):
        for j in range(2):
            i = t * 2 + j
            gather_a(j).wait()
            gather_b(j).start()
            sums_a = reduce_rows(bufs[0], LA)
            for c in range(_NCH):
                out_v[i, pl.ds(_OFFS[c], 16)] = sums_a[c]
            gather_b(j).wait()

            @pl.when(i + 1 < BPW)
            def _():
                wait_idx(i + 1, 1 - j)
                gather_a(1 - j).start()

            @pl.when(i + 2 < BPW)
            def _():
                fire_idx(i + 2, j)

            sums_b = reduce_rows(bufs[1], LB)
            for c in range(_NCH):
                out_v[i, pl.ds(_OFFS[c], 16)] = (
                    out_v[i, pl.ds(_OFFS[c], 16)] + sums_b[c]) * scale

    pltpu.sync_copy(out_v, out_hbm.at[pl.ds(base, BPW)])


_sc_pool = functools.partial(
    pl.kernel,
    out_type=jax.ShapeDtypeStruct((B, EMB), jnp.float32),
    mesh=plsc.VectorSubcoreMesh(core_axis_name="c", subcore_axis_name="s"),
    scratch_types=(
        [pltpu.VMEM((L,), jnp.int32) for _ in range(2)]
        + [pltpu.VMEM((LA, EMB), jnp.float32) for _ in range(2)]
        + [pltpu.VMEM((BPW, EMB), jnp.float32)]
        + [pltpu.SemaphoreType.DMA for _ in range(4)]
    ),
    compiler_params=pltpu.CompilerParams(use_tc_tiling_on_sc=False),
)(_sc_pool_body)


def _mlp_body(x_ref, w1_ref, b1_ref, w2_ref, b2_ref, w3_ref, b3_ref, o_ref):
    h = jnp.dot(x_ref[...], w1_ref[...], preferred_element_type=jnp.float32)
    h = jnp.maximum(h + b1_ref[...], 0.0).astype(jnp.bfloat16)
    h = jnp.dot(h, w2_ref[...], preferred_element_type=jnp.float32)
    h = jnp.maximum(h + b2_ref[...], 0.0).astype(jnp.bfloat16)
    logits = jnp.dot(h, w3_ref[...], preferred_element_type=jnp.float32)
    logits = logits + b3_ref[...]
    m = jnp.max(logits, axis=1, keepdims=True)
    lse = jnp.log(jnp.sum(jnp.exp(logits - m), axis=1, keepdims=True)) + m
    o_ref[...] = logits - lse


BM = 512

_mlp = pl.pallas_call(
    _mlp_body,
    grid=(B // BM,),
    in_specs=[
        pl.BlockSpec((BM, EMB), lambda i: (i, 0)),
        pl.BlockSpec((EMB, HID), lambda i: (0, 0)),
        pl.BlockSpec((1, HID), lambda i: (0, 0)),
        pl.BlockSpec((HID, HID), lambda i: (0, 0)),
        pl.BlockSpec((1, HID), lambda i: (0, 0)),
        pl.BlockSpec((HID, 2), lambda i: (0, 0)),
        pl.BlockSpec((1, 2), lambda i: (0, 0)),
    ],
    out_specs=pl.BlockSpec((BM, 2), lambda i: (i, 0)),
    out_shape=jax.ShapeDtypeStruct((B, 2), jnp.float32),
    compiler_params=pltpu.CompilerParams(
        dimension_semantics=("arbitrary",)),
)


def kernel(x, emb, W1, b1, W2, b2, W3, b3):
    pooled = _sc_pool(x, emb)
    return _mlp(pooled.astype(jnp.bfloat16),
                W1.astype(jnp.bfloat16), b1.reshape(1, HID),
                W2.astype(jnp.bfloat16), b2.reshape(1, HID),
                W3.astype(jnp.bfloat16), b3.reshape(1, 2))


# speedup vs baseline: 1.8407x; 1.0679x over previous
import functools

import jax
import jax.numpy as jnp
from jax import lax
from jax.experimental import pallas as pl
from jax.experimental.pallas import tpu as pltpu
from jax.experimental.pallas import tpu_sc as plsc

VOCAB = 100000
EMB = 300
HID = 4096
B = 4096
L = 200

NC = 2
NS = 16
NW = NC * NS
BPW = B // NW
LA = 104
LB = L - LA

EMBP = 384

_OFFS = tuple(16 * c for c in range(19))
_NCH = len(_OFFS)
_PAD_OFFS = tuple(16 * c for c in range(19, EMBP // 16))


def _sc_pool_body(x_hbm, emb_hbm, out_hbm, *refs):
    idxb = refs[0:2]
    bufs = refs[2:4]
    out_v = refs[4]
    gsem = refs[5:7]
    isem = refs[7:9]

    wid = lax.axis_index("s") * NC + lax.axis_index("c")
    base = wid * BPW
    zero = jnp.zeros((16,), jnp.float32)
    scale = jnp.full((16,), 1.0 / L, jnp.float32)

    @pl.loop(0, BPW)
    def _(r):
        for off in _PAD_OFFS:
            out_v[r, pl.ds(off, 16)] = zero

    def fire_idx(i, slot):
        pltpu.async_copy(x_hbm.at[pl.ds((base + i) * L, L)],
                         idxb[slot], isem[slot])

    def wait_idx(i, slot):
        pltpu.make_async_copy(x_hbm.at[pl.ds((base + i) * L, L)],
                              idxb[slot], isem[slot]).wait()

    def gather_a(slot):
        return pltpu.make_async_copy(
            emb_hbm.at[idxb[slot].at[pl.ds(0, LA)]], bufs[0], gsem[0])

    def gather_b(slot):
        return pltpu.make_async_copy(
            emb_hbm.at[idxb[slot].at[pl.ds(LA, LB)]], bufs[1], gsem[1])

    def reduce_rows(buf, n):
        @pl.loop(0, n, init_carry=(zero,) * _NCH)
        def sums(r, carry):
            return tuple(carry[c] + buf[r, pl.ds(_OFFS[c], 16)]
                         for c in range(_NCH))
        return sums

    fire_idx(0, 0)
    fire_idx(1, 1)
    wait_idx(0, 0)
    gather_a(0).start()

    @pl.loop(0, BPW // 2)
    def _(t):
        for j in range(2):
            i = t * 2 + j
            gather_a(j).wait()
            gather_b(j).start()
            sums_a = reduce_rows(bufs[0], LA)
            for c in range(_NCH):
                out_v[i, pl.ds(_OFFS[c], 16)] = sums_a[c]
            gather_b(j).wait()

            @pl.when(i + 1 < BPW)
            def _():
                wait_idx(i + 1, 1 - j)
                gather_a(1 - j).start()

            @pl.when(i + 2 < BPW)
            def _():
                fire_idx(i + 2, j)

            sums_b = reduce_rows(bufs[1], LB)
            for c in range(_NCH):
                out_v[i, pl.ds(_OFFS[c], 16)] = (
                    out_v[i, pl.ds(_OFFS[c], 16)] + sums_b[c]) * scale

    pltpu.sync_copy(out_v, out_hbm.at[pl.ds(base, BPW)])


_sc_pool = functools.partial(
    pl.kernel,
    out_type=jax.ShapeDtypeStruct((B, EMBP), jnp.float32),
    mesh=plsc.VectorSubcoreMesh(core_axis_name="c", subcore_axis_name="s"),
    scratch_types=(
        [pltpu.VMEM((L,), jnp.int32) for _ in range(2)]
        + [pltpu.VMEM((LA, EMBP), jnp.float32),
           pltpu.VMEM((LB, EMBP), jnp.float32)]
        + [pltpu.VMEM((BPW, EMBP), jnp.float32)]
        + [pltpu.SemaphoreType.DMA for _ in range(4)]
    ),
)(_sc_pool_body)


def _mlp_body(x_ref, w1_ref, b1_ref, w2_ref, b2_ref, w3_ref, b3_ref, o_ref):
    h = jnp.dot(x_ref[...], w1_ref[...], preferred_element_type=jnp.float32)
    h = jnp.maximum(h + b1_ref[...], 0.0).astype(jnp.bfloat16)
    h = jnp.dot(h, w2_ref[...], preferred_element_type=jnp.float32)
    h = jnp.maximum(h + b2_ref[...], 0.0).astype(jnp.bfloat16)
    logits = jnp.dot(h, w3_ref[...], preferred_element_type=jnp.float32)
    logits = logits + b3_ref[...]
    m = jnp.max(logits, axis=1, keepdims=True)
    lse = jnp.log(jnp.sum(jnp.exp(logits - m), axis=1, keepdims=True)) + m
    o_ref[...] = logits - lse


BM = 512

_mlp = pl.pallas_call(
    _mlp_body,
    grid=(B // BM,),
    in_specs=[
        pl.BlockSpec((BM, EMBP), lambda i: (i, 0)),
        pl.BlockSpec((EMBP, HID), lambda i: (0, 0)),
        pl.BlockSpec((1, HID), lambda i: (0, 0)),
        pl.BlockSpec((HID, HID), lambda i: (0, 0)),
        pl.BlockSpec((1, HID), lambda i: (0, 0)),
        pl.BlockSpec((HID, 2), lambda i: (0, 0)),
        pl.BlockSpec((1, 2), lambda i: (0, 0)),
    ],
    out_specs=pl.BlockSpec((BM, 2), lambda i: (i, 0)),
    out_shape=jax.ShapeDtypeStruct((B, 2), jnp.float32),
    compiler_params=pltpu.CompilerParams(
        dimension_semantics=("arbitrary",)),
)


def kernel(x, emb, W1, b1, W2, b2, W3, b3):
    emb_p = jnp.pad(emb, ((0, 0), (0, EMBP - EMB)))
    w1_p = jnp.pad(W1, ((0, EMBP - EMB), (0, 0)))
    pooled = _sc_pool(x.reshape(-1), emb_p)
    return _mlp(pooled.astype(jnp.bfloat16),
                w1_p.astype(jnp.bfloat16), b1.reshape(1, HID),
                W2.astype(jnp.bfloat16), b2.reshape(1, HID),
                W3.astype(jnp.bfloat16), b3.reshape(1, 2))


# speedup vs baseline: 2.5432x; 1.3816x over previous
import functools

import jax
import jax.numpy as jnp
from jax import lax
from jax.experimental import pallas as pl
from jax.experimental.pallas import tpu as pltpu
from jax.experimental.pallas import tpu_sc as plsc

VOCAB = 100000
EMB = 300
HID = 4096
B = 4096
L = 200

NC = 2
NS = 16
NW = NC * NS
BPW = B // NW
LA = 104
LB = L - LA

EMBP = 384

_OFFS = tuple(16 * c for c in range(19))
_NCH = len(_OFFS)
_PAD_OFFS = tuple(16 * c for c in range(19, EMBP // 16))


def _sc_pool_body(x_hbm, emb_hbm, out_hbm, *refs):
    idxb = refs[0:2]
    bufs = refs[2:4]
    out_v = refs[4]
    gsem = refs[5:7]
    isem = refs[7:9]

    wid = lax.axis_index("s") * NC + lax.axis_index("c")
    base = wid * BPW
    zero = jnp.zeros((16,), jnp.float32)
    scale = jnp.full((16,), 1.0 / L, jnp.float32)

    @pl.loop(0, BPW)
    def _(r):
        for off in _PAD_OFFS:
            out_v[r, pl.ds(off, 16)] = zero

    def fire_idx(i, slot):
        pltpu.async_copy(x_hbm.at[pl.ds((base + i) * L, L)],
                         idxb[slot], isem[slot])

    def wait_idx(i, slot):
        pltpu.make_async_copy(x_hbm.at[pl.ds((base + i) * L, L)],
                              idxb[slot], isem[slot]).wait()

    def gather_a(slot):
        return pltpu.make_async_copy(
            emb_hbm.at[idxb[slot].at[pl.ds(0, LA)]], bufs[0], gsem[0])

    def gather_b(slot):
        return pltpu.make_async_copy(
            emb_hbm.at[idxb[slot].at[pl.ds(LA, LB)]], bufs[1], gsem[1])

    def reduce_rows(buf, n):
        @pl.loop(0, n, init_carry=(zero,) * _NCH)
        def sums(r, carry):
            return tuple(carry[c] + buf[r, pl.ds(_OFFS[c], 16)]
                         for c in range(_NCH))
        return sums

    fire_idx(0, 0)
    fire_idx(1, 1)
    wait_idx(0, 0)
    gather_a(0).start()

    @pl.loop(0, BPW // 2)
    def _(t):
        for j in range(2):
            i = t * 2 + j
            gather_a(j).wait()
            gather_b(j).start()
            sums_a = reduce_rows(bufs[0], LA)
            for c in range(_NCH):
                out_v[i, pl.ds(_OFFS[c], 16)] = sums_a[c]
            gather_b(j).wait()

            @pl.when(i + 1 < BPW)
            def _():
                wait_idx(i + 1, 1 - j)
                gather_a(1 - j).start()

            @pl.when(i + 2 < BPW)
            def _():
                fire_idx(i + 2, j)

            sums_b = reduce_rows(bufs[1], LB)
            for c in range(_NCH):
                out_v[i, pl.ds(_OFFS[c], 16)] = (
                    out_v[i, pl.ds(_OFFS[c], 16)] + sums_b[c]) * scale

    pltpu.sync_copy(out_v, out_hbm.at[pl.ds(base, BPW)])


_sc_pool = functools.partial(
    pl.kernel,
    out_type=jax.ShapeDtypeStruct((B, EMBP), jnp.float32),
    mesh=plsc.VectorSubcoreMesh(core_axis_name="c", subcore_axis_name="s"),
    scratch_types=(
        [pltpu.VMEM((L,), jnp.int32) for _ in range(2)]
        + [pltpu.VMEM((LA, EMBP), jnp.float32),
           pltpu.VMEM((LB, EMBP), jnp.float32)]
        + [pltpu.VMEM((BPW, EMBP), jnp.float32)]
        + [pltpu.SemaphoreType.DMA for _ in range(4)]
    ),
)(_sc_pool_body)


def _trans_body(a_ref, i_ref, o_ref):
    o_ref[...] = lax.dot_general(
        a_ref[...], i_ref[...],
        dimension_numbers=(((0,), (0,)), ((), ())),
        preferred_element_type=jnp.float32)


_TV = 512

_trans = pl.pallas_call(
    _trans_body,
    grid=((VOCAB + _TV - 1) // _TV,),
    in_specs=[
        pl.BlockSpec((EMB, _TV), lambda i: (0, i)),
        pl.BlockSpec((EMB, EMBP), lambda i: (0, 0)),
    ],
    out_specs=pl.BlockSpec((_TV, EMBP), lambda i: (i, 0)),
    out_shape=jax.ShapeDtypeStruct((VOCAB, EMBP), jnp.float32),
    compiler_params=pltpu.CompilerParams(
        dimension_semantics=("arbitrary",)),
)


def _mlp_body(x_ref, w1_ref, b1_ref, w2_ref, b2_ref, w3_ref, b3_ref, o_ref):
    h = jnp.dot(x_ref[...], w1_ref[...], preferred_element_type=jnp.float32)
    h = jnp.maximum(h + b1_ref[...], 0.0).astype(jnp.bfloat16)
    h = jnp.dot(h, w2_ref[...], preferred_element_type=jnp.float32)
    h = jnp.maximum(h + b2_ref[...], 0.0).astype(jnp.bfloat16)
    logits = jnp.dot(h, w3_ref[...], preferred_element_type=jnp.float32)
    logits = logits + b3_ref[...]
    m = jnp.max(logits, axis=1, keepdims=True)
    lse = jnp.log(jnp.sum(jnp.exp(logits - m), axis=1, keepdims=True)) + m
    o_ref[...] = logits - lse


BM = 512

_mlp = pl.pallas_call(
    _mlp_body,
    grid=(B // BM,),
    in_specs=[
        pl.BlockSpec((BM, EMBP), lambda i: (i, 0)),
        pl.BlockSpec((EMBP, HID), lambda i: (0, 0)),
        pl.BlockSpec((1, HID), lambda i: (0, 0)),
        pl.BlockSpec((HID, HID), lambda i: (0, 0)),
        pl.BlockSpec((1, HID), lambda i: (0, 0)),
        pl.BlockSpec((HID, 2), lambda i: (0, 0)),
        pl.BlockSpec((1, 2), lambda i: (0, 0)),
    ],
    out_specs=pl.BlockSpec((BM, 2), lambda i: (i, 0)),
    out_shape=jax.ShapeDtypeStruct((B, 2), jnp.float32),
    compiler_params=pltpu.CompilerParams(
        dimension_semantics=("arbitrary",)),
)


def kernel(x, emb, W1, b1, W2, b2, W3, b3):
    emb_p = _trans(emb.T, jnp.eye(EMB, EMBP, dtype=jnp.float32))
    w1_p = jnp.pad(W1, ((0, EMBP - EMB), (0, 0)))
    pooled = _sc_pool(x.reshape(-1), emb_p)
    return _mlp(pooled.astype(jnp.bfloat16),
                w1_p.astype(jnp.bfloat16), b1.reshape(1, HID),
                W2.astype(jnp.bfloat16), b2.reshape(1, HID),
                W3.astype(jnp.bfloat16), b3.reshape(1, 2))


# speedup vs baseline: 2.9231x; 1.1494x over previous
import functools

import jax
import jax.numpy as jnp
from jax import lax
from jax.experimental import pallas as pl
from jax.experimental.pallas import tpu as pltpu
from jax.experimental.pallas import tpu_sc as plsc

VOCAB = 100000
EMB = 300
HID = 4096
B = 4096
L = 200

NC = 2
NS = 16
NW = NC * NS
BPW = B // NW
LA = 104
LB = L - LA

EMBP = 384
HALF = EMBP // 2
PACKW = 256

_NCHP = HALF // 16


def _sc_pool_body(x_hbm, emb_hbm, out_hbm, *refs):
    idxb = refs[0:2]
    bufs = refs[2:4]
    out_v = refs[4]
    gsem = refs[5:7]
    isem = refs[7:9]

    wid = lax.axis_index("s") * NC + lax.axis_index("c")
    base = wid * BPW
    zero = jnp.zeros((16,), jnp.float32)
    scale = jnp.full((16,), 1.0 / L, jnp.float32)

    def fire_idx(i, slot):
        pltpu.async_copy(x_hbm.at[pl.ds((base + i) * L, L)],
                         idxb[slot], isem[slot])

    def wait_idx(i, slot):
        pltpu.make_async_copy(x_hbm.at[pl.ds((base + i) * L, L)],
                              idxb[slot], isem[slot]).wait()

    def gather_a(slot):
        return pltpu.make_async_copy(
            emb_hbm.at[idxb[slot].at[pl.ds(0, LA)]], bufs[0], gsem[0])

    def gather_b(slot):
        return pltpu.make_async_copy(
            emb_hbm.at[idxb[slot].at[pl.ds(LA, LB)]], bufs[1], gsem[1])

    def reduce_rows(buf, n):
        @pl.loop(0, n, init_carry=(zero,) * (2 * _NCHP))
        def sums(r, carry):
            vals = []
            for c in range(_NCHP):
                w = buf[r, pl.ds(16 * c, 16)]
                lo, hi = plsc.unpack(plsc.bitcast(w, jnp.bfloat16),
                                     format=plsc.PackFormat.INTERLEAVED)
                vals.append(carry[2 * c] + lo)
                vals.append(carry[2 * c + 1] + hi)
            return tuple(vals)
        return sums

    fire_idx(0, 0)
    fire_idx(1, 1)
    wait_idx(0, 0)
    gather_a(0).start()

    @pl.loop(0, BPW // 2)
    def _(t):
        for j in range(2):
            i = t * 2 + j
            gather_a(j).wait()
            gather_b(j).start()
            sums_a = reduce_rows(bufs[0], LA)
            for c in range(_NCHP):
                out_v[i, pl.ds(16 * c, 16)] = sums_a[2 * c]
                out_v[i, pl.ds(HALF + 16 * c, 16)] = sums_a[2 * c + 1]
            gather_b(j).wait()

            @pl.when(i + 1 < BPW)
            def _():
                wait_idx(i + 1, 1 - j)
                gather_a(1 - j).start()

            @pl.when(i + 2 < BPW)
            def _():
                fire_idx(i + 2, j)

            sums_b = reduce_rows(bufs[1], LB)
            for c in range(_NCHP):
                out_v[i, pl.ds(16 * c, 16)] = (
                    out_v[i, pl.ds(16 * c, 16)] + sums_b[2 * c]) * scale
                out_v[i, pl.ds(HALF + 16 * c, 16)] = (
                    out_v[i, pl.ds(HALF + 16 * c, 16)]
                    + sums_b[2 * c + 1]) * scale

    pltpu.sync_copy(out_v, out_hbm.at[pl.ds(base, BPW)])


_sc_pool = functools.partial(
    pl.kernel,
    out_type=jax.ShapeDtypeStruct((B, EMBP), jnp.float32),
    mesh=plsc.VectorSubcoreMesh(core_axis_name="c", subcore_axis_name="s"),
    scratch_types=(
        [pltpu.VMEM((L,), jnp.int32) for _ in range(2)]
        + [pltpu.VMEM((LA, PACKW), jnp.float32),
           pltpu.VMEM((LB, PACKW), jnp.float32)]
        + [pltpu.VMEM((BPW, EMBP), jnp.float32)]
        + [pltpu.SemaphoreType.DMA for _ in range(4)]
    ),
    compiler_params=pltpu.CompilerParams(needs_layout_passes=False),
)(_sc_pool_body)


def _trans_body(a_ref, i_ref, o_ref):
    res = lax.dot_general(
        a_ref[...], i_ref[...],
        dimension_numbers=(((0,), (0,)), ((), ())),
        preferred_element_type=jnp.float32)
    resb = res.astype(jnp.bfloat16)
    lo = lax.bitcast_convert_type(resb[:, :HALF], jnp.uint16)
    hi = lax.bitcast_convert_type(resb[:, HALF:], jnp.uint16)
    packed = lo.astype(jnp.uint32) | (hi.astype(jnp.uint32) << 16)
    o_ref[:, :HALF] = lax.bitcast_convert_type(packed, jnp.float32)
    o_ref[:, HALF:] = jnp.zeros((o_ref.shape[0], PACKW - HALF), jnp.float32)


_TV = 512

_trans = pl.pallas_call(
    _trans_body,
    grid=((VOCAB + _TV - 1) // _TV,),
    in_specs=[
        pl.BlockSpec((EMB, _TV), lambda i: (0, i)),
        pl.BlockSpec((EMB, EMBP), lambda i: (0, 0)),
    ],
    out_specs=pl.BlockSpec((_TV, PACKW), lambda i: (i, 0)),
    out_shape=jax.ShapeDtypeStruct((VOCAB, PACKW), jnp.float32),
    compiler_params=pltpu.CompilerParams(
        dimension_semantics=("arbitrary",)),
)


def _mlp_body(x_ref, w1_ref, b1_ref, w2_ref, b2_ref, w3_ref, b3_ref, o_ref):
    h = jnp.dot(x_ref[...], w1_ref[...], preferred_element_type=jnp.float32)
    h = jnp.maximum(h + b1_ref[...], 0.0).astype(jnp.bfloat16)
    h = jnp.dot(h, w2_ref[...], preferred_element_type=jnp.float32)
    h = jnp.maximum(h + b2_ref[...], 0.0).astype(jnp.bfloat16)
    logits = jnp.dot(h, w3_ref[...], preferred_element_type=jnp.float32)
    logits = logits + b3_ref[...]
    m = jnp.max(logits, axis=1, keepdims=True)
    lse = jnp.log(jnp.sum(jnp.exp(logits - m), axis=1, keepdims=True)) + m
    o_ref[...] = logits - lse


BM = 512

_mlp = pl.pallas_call(
    _mlp_body,
    grid=(B // BM,),
    in_specs=[
        pl.BlockSpec((BM, EMBP), lambda i: (i, 0)),
        pl.BlockSpec((EMBP, HID), lambda i: (0, 0)),
        pl.BlockSpec((1, HID), lambda i: (0, 0)),
        pl.BlockSpec((HID, HID), lambda i: (0, 0)),
        pl.BlockSpec((1, HID), lambda i: (0, 0)),
        pl.BlockSpec((HID, 2), lambda i: (0, 0)),
        pl.BlockSpec((1, 2), lambda i: (0, 0)),
    ],
    out_specs=pl.BlockSpec((BM, 2), lambda i: (i, 0)),
    out_shape=jax.ShapeDtypeStruct((B, 2), jnp.float32),
    compiler_params=pltpu.CompilerParams(
        dimension_semantics=("arbitrary",)),
)


def kernel(x, emb, W1, b1, W2, b2, W3, b3):
    emb_p = _trans(emb.T, jnp.eye(EMB, EMBP, dtype=jnp.float32))
    w1_p = jnp.pad(W1, ((0, EMBP - EMB), (0, 0)))
    pooled = _sc_pool(x.reshape(-1), emb_p)
    return _mlp(pooled.astype(jnp.bfloat16),
                w1_p.astype(jnp.bfloat16), b1.reshape(1, HID),
                W2.astype(jnp.bfloat16), b2.reshape(1, HID),
                W3.astype(jnp.bfloat16), b3.reshape(1, 2))


# speedup vs baseline: 3.1514x; 1.0781x over previous
import functools

import jax
import jax.numpy as jnp
from jax import lax
from jax.experimental import pallas as pl
from jax.experimental.pallas import tpu as pltpu
from jax.experimental.pallas import tpu_sc as plsc

VOCAB = 100000
EMB = 300
HID = 4096
B = 4096
L = 200

NC = 2
NS = 16
NW = NC * NS
NCHUNK = 2
BC = B // NCHUNK
BPW = BC // NW
LA = 104
LB = L - LA

EMBP = 384
HALF = EMBP // 2
PACKW = 256

_NCHP = HALF // 16


def _sc_pool_body(x_hbm, emb_hbm, out_hbm, *refs):
    idxb = refs[0:2]
    bufs = refs[2:4]
    out_v = refs[4]
    gsem = refs[5:7]
    isem = refs[7:9]

    wid = lax.axis_index("s") * NC + lax.axis_index("c")
    base = wid * BPW
    zero = jnp.zeros((16,), jnp.float32)
    scale = jnp.full((16,), 1.0 / L, jnp.float32)

    def fire_idx(i, slot):
        pltpu.async_copy(x_hbm.at[pl.ds((base + i) * L, L)],
                         idxb[slot], isem[slot])

    def wait_idx(i, slot):
        pltpu.make_async_copy(x_hbm.at[pl.ds((base + i) * L, L)],
                              idxb[slot], isem[slot]).wait()

    def gather_a(slot):
        return pltpu.make_async_copy(
            emb_hbm.at[idxb[slot].at[pl.ds(0, LA)]], bufs[0], gsem[0])

    def gather_b(slot):
        return pltpu.make_async_copy(
            emb_hbm.at[idxb[slot].at[pl.ds(LA, LB)]], bufs[1], gsem[1])

    def reduce_rows(buf, n):
        @pl.loop(0, n, init_carry=(zero,) * (2 * _NCHP))
        def sums(r, carry):
            vals = []
            for c in range(_NCHP):
                w = buf[r, pl.ds(16 * c, 16)]
                lo, hi = plsc.unpack(plsc.bitcast(w, jnp.bfloat16),
                                     format=plsc.PackFormat.INTERLEAVED)
                vals.append(carry[2 * c] + lo)
                vals.append(carry[2 * c + 1] + hi)
            return tuple(vals)
        return sums

    fire_idx(0, 0)
    fire_idx(1, 1)
    wait_idx(0, 0)
    gather_a(0).start()

    @pl.loop(0, BPW // 2)
    def _(t):
        for j in range(2):
            i = t * 2 + j
            gather_a(j).wait()
            gather_b(j).start()
            sums_a = reduce_rows(bufs[0], LA)
            for c in range(_NCHP):
                out_v[i, pl.ds(16 * c, 16)] = sums_a[2 * c]
                out_v[i, pl.ds(HALF + 16 * c, 16)] = sums_a[2 * c + 1]
            gather_b(j).wait()

            @pl.when(i + 1 < BPW)
            def _():
                wait_idx(i + 1, 1 - j)
                gather_a(1 - j).start()

            @pl.when(i + 2 < BPW)
            def _():
                fire_idx(i + 2, j)

            sums_b = reduce_rows(bufs[1], LB)
            for c in range(_NCHP):
                out_v[i, pl.ds(16 * c, 16)] = (
                    out_v[i, pl.ds(16 * c, 16)] + sums_b[2 * c]) * scale
                out_v[i, pl.ds(HALF + 16 * c, 16)] = (
                    out_v[i, pl.ds(HALF + 16 * c, 16)]
                    + sums_b[2 * c + 1]) * scale

    pltpu.sync_copy(out_v, out_hbm.at[pl.ds(base, BPW)])


_sc_pool = functools.partial(
    pl.kernel,
    out_type=jax.ShapeDtypeStruct((BC, EMBP), jnp.float32),
    mesh=plsc.VectorSubcoreMesh(core_axis_name="c", subcore_axis_name="s"),
    scratch_types=(
        [pltpu.VMEM((L,), jnp.int32) for _ in range(2)]
        + [pltpu.VMEM((LA, PACKW), jnp.float32),
           pltpu.VMEM((LB, PACKW), jnp.float32)]
        + [pltpu.VMEM((BPW, EMBP), jnp.float32)]
        + [pltpu.SemaphoreType.DMA for _ in range(4)]
    ),
    compiler_params=pltpu.CompilerParams(needs_layout_passes=False),
)(_sc_pool_body)


def _trans_body(a_ref, i_ref, o_ref):
    res = lax.dot_general(
        a_ref[...].astype(jnp.bfloat16), i_ref[...],
        dimension_numbers=(((0,), (0,)), ((), ())),
        preferred_element_type=jnp.float32)
    resb = res.astype(jnp.bfloat16)
    lo = lax.bitcast_convert_type(resb[:, :HALF], jnp.uint16)
    hi = lax.bitcast_convert_type(resb[:, HALF:], jnp.uint16)
    packed = lo.astype(jnp.uint32) | (hi.astype(jnp.uint32) << 16)
    o_ref[:, :HALF] = lax.bitcast_convert_type(packed, jnp.float32)
    o_ref[:, HALF:] = jnp.zeros((o_ref.shape[0], PACKW - HALF), jnp.float32)


_TV = 512

_trans = pl.pallas_call(
    _trans_body,
    grid=((VOCAB + _TV - 1) // _TV,),
    in_specs=[
        pl.BlockSpec((EMB, _TV), lambda i: (0, i)),
        pl.BlockSpec((EMB, EMBP), lambda i: (0, 0)),
    ],
    out_specs=pl.BlockSpec((_TV, PACKW), lambda i: (i, 0)),
    out_shape=jax.ShapeDtypeStruct((VOCAB, PACKW), jnp.float32),
    compiler_params=pltpu.CompilerParams(
        dimension_semantics=("arbitrary",)),
)


def _mlp_body(x_ref, w1_ref, b1_ref, w2_ref, b2_ref, w3_ref, b3_ref, o_ref):
    h = jnp.dot(x_ref[...], w1_ref[...], preferred_element_type=jnp.float32)
    h = jnp.maximum(h + b1_ref[...], 0.0).astype(jnp.bfloat16)
    h = jnp.dot(h, w2_ref[...], preferred_element_type=jnp.float32)
    h = jnp.maximum(h + b2_ref[...], 0.0).astype(jnp.bfloat16)
    logits = jnp.dot(h, w3_ref[...], preferred_element_type=jnp.float32)
    logits = logits + b3_ref[...]
    m = jnp.max(logits, axis=1, keepdims=True)
    lse = jnp.log(jnp.sum(jnp.exp(logits - m), axis=1, keepdims=True)) + m
    o_ref[...] = logits - lse


BM = 512

_mlp = pl.pallas_call(
    _mlp_body,
    grid=(BC // BM,),
    in_specs=[
        pl.BlockSpec((BM, EMBP), lambda i: (i, 0)),
        pl.BlockSpec((EMBP, HID), lambda i: (0, 0)),
        pl.BlockSpec((1, HID), lambda i: (0, 0)),
        pl.BlockSpec((HID, HID), lambda i: (0, 0)),
        pl.BlockSpec((1, HID), lambda i: (0, 0)),
        pl.BlockSpec((HID, 2), lambda i: (0, 0)),
        pl.BlockSpec((1, 2), lambda i: (0, 0)),
    ],
    out_specs=pl.BlockSpec((BM, 2), lambda i: (i, 0)),
    out_shape=jax.ShapeDtypeStruct((BC, 2), jnp.float32),
    compiler_params=pltpu.CompilerParams(
        dimension_semantics=("arbitrary",)),
)


def kernel(x, emb, W1, b1, W2, b2, W3, b3):
    emb_p = _trans(emb.T, jnp.eye(EMB, EMBP, dtype=jnp.bfloat16))
    w1_p = jnp.pad(W1, ((0, EMBP - EMB), (0, 0)))
    w1b = w1_p.astype(jnp.bfloat16)
    w2b = W2.astype(jnp.bfloat16)
    w3b = W3.astype(jnp.bfloat16)
    b1r = b1.reshape(1, HID)
    b2r = b2.reshape(1, HID)
    b3r = b3.reshape(1, 2)
    xf = x.reshape(-1)
    pooled = [_sc_pool(lax.dynamic_slice_in_dim(xf, k * BC * L, BC * L),
                       emb_p)
              for k in range(NCHUNK)]
    outs = [_mlp(p.astype(jnp.bfloat16), w1b, b1r, w2b, b2r, w3b, b3r)
            for p in pooled]
    return jnp.concatenate(outs, axis=0)


# speedup vs baseline: 3.2439x; 1.0293x over previous
import functools

import jax
import jax.numpy as jnp
from jax import lax
from jax.experimental import pallas as pl
from jax.experimental.pallas import tpu as pltpu
from jax.experimental.pallas import tpu_sc as plsc

VOCAB = 100000
EMB = 300
HID = 4096
B = 4096
L = 200

NC = 2
NS = 16
NW = NC * NS
NCHUNK = 4
BC = B // NCHUNK
BPW = BC // NW
LA = 104
LB = L - LA

EMBP = 384
HALF = EMBP // 2
PACKW = 256

_NCHP = HALF // 16


def _sc_pool_body(x_hbm, emb_hbm, out_hbm, *refs):
    idxb = refs[0:2]
    bufs = refs[2:4]
    out_v = refs[4]
    gsem = refs[5:7]
    isem = refs[7:9]

    wid = lax.axis_index("s") * NC + lax.axis_index("c")
    base = wid * BPW
    zero = jnp.zeros((16,), jnp.float32)
    scale = jnp.full((16,), 1.0 / L, jnp.float32)

    def fire_idx(i, slot):
        pltpu.async_copy(x_hbm.at[pl.ds((base + i) * L, L)],
                         idxb[slot], isem[slot])

    def wait_idx(i, slot):
        pltpu.make_async_copy(x_hbm.at[pl.ds((base + i) * L, L)],
                              idxb[slot], isem[slot]).wait()

    def gather_a(slot):
        return pltpu.make_async_copy(
            emb_hbm.at[idxb[slot].at[pl.ds(0, LA)]], bufs[0], gsem[0])

    def gather_b(slot):
        return pltpu.make_async_copy(
            emb_hbm.at[idxb[slot].at[pl.ds(LA, LB)]], bufs[1], gsem[1])

    def reduce_rows(buf, n):
        @pl.loop(0, n, init_carry=(zero,) * (2 * _NCHP))
        def sums(r, carry):
            vals = []
            for c in range(_NCHP):
                w = buf[r, pl.ds(16 * c, 16)]
                lo, hi = plsc.unpack(plsc.bitcast(w, jnp.bfloat16),
                                     format=plsc.PackFormat.INTERLEAVED)
                vals.append(carry[2 * c] + lo)
                vals.append(carry[2 * c + 1] + hi)
            return tuple(vals)
        return sums

    fire_idx(0, 0)
    fire_idx(1, 1)
    wait_idx(0, 0)
    gather_a(0).start()

    @pl.loop(0, BPW // 2)
    def _(t):
        for j in range(2):
            i = t * 2 + j
            gather_a(j).wait()
            gather_b(j).start()
            sums_a = reduce_rows(bufs[0], LA)
            for c in range(_NCHP):
                out_v[i, pl.ds(16 * c, 16)] = sums_a[2 * c]
                out_v[i, pl.ds(HALF + 16 * c, 16)] = sums_a[2 * c + 1]
            gather_b(j).wait()

            @pl.when(i + 1 < BPW)
            def _():
                wait_idx(i + 1, 1 - j)
                gather_a(1 - j).start()

            @pl.when(i + 2 < BPW)
            def _():
                fire_idx(i + 2, j)

            sums_b = reduce_rows(bufs[1], LB)
            for c in range(_NCHP):
                out_v[i, pl.ds(16 * c, 16)] = (
                    out_v[i, pl.ds(16 * c, 16)] + sums_b[2 * c]) * scale
                out_v[i, pl.ds(HALF + 16 * c, 16)] = (
                    out_v[i, pl.ds(HALF + 16 * c, 16)]
                    + sums_b[2 * c + 1]) * scale

    pltpu.sync_copy(out_v, out_hbm.at[pl.ds(base, BPW)])


_sc_pool = functools.partial(
    pl.kernel,
    out_type=jax.ShapeDtypeStruct((BC, EMBP), jnp.float32),
    mesh=plsc.VectorSubcoreMesh(core_axis_name="c", subcore_axis_name="s"),
    scratch_types=(
        [pltpu.VMEM((L,), jnp.int32) for _ in range(2)]
        + [pltpu.VMEM((LA, PACKW), jnp.float32),
           pltpu.VMEM((LB, PACKW), jnp.float32)]
        + [pltpu.VMEM((BPW, EMBP), jnp.float32)]
        + [pltpu.SemaphoreType.DMA for _ in range(4)]
    ),
    compiler_params=pltpu.CompilerParams(needs_layout_passes=False),
)(_sc_pool_body)


def _trans_body(a_ref, i_ref, o_ref):
    res = lax.dot_general(
        a_ref[...].astype(jnp.bfloat16), i_ref[...],
        dimension_numbers=(((0,), (0,)), ((), ())),
        preferred_element_type=jnp.float32)
    bits = lax.bitcast_convert_type(res, jnp.uint32) + jnp.uint32(0x8000)
    lo = bits[:, :HALF] >> 16
    hi = bits[:, HALF:] & jnp.uint32(0xFFFF0000)
    o_ref[:, :HALF] = lax.bitcast_convert_type(lo | hi, jnp.float32)
    o_ref[:, HALF:] = jnp.zeros((o_ref.shape[0], PACKW - HALF), jnp.float32)


_TV = 512

_trans = pl.pallas_call(
    _trans_body,
    grid=((VOCAB + _TV - 1) // _TV,),
    in_specs=[
        pl.BlockSpec((EMB, _TV), lambda i: (0, i)),
        pl.BlockSpec((EMB, EMBP), lambda i: (0, 0)),
    ],
    out_specs=pl.BlockSpec((_TV, PACKW), lambda i: (i, 0)),
    out_shape=jax.ShapeDtypeStruct((VOCAB, PACKW), jnp.float32),
    compiler_params=pltpu.CompilerParams(
        dimension_semantics=("arbitrary",)),
)


def _mlp_body(x_ref, w1_ref, b1_ref, w2_ref, b2_ref, w3_ref, b3_ref, o_ref):
    h = jnp.dot(x_ref[...], w1_ref[...], preferred_element_type=jnp.float32)
    h = jnp.maximum(h + b1_ref[...], 0.0).astype(jnp.bfloat16)
    h = jnp.dot(h, w2_ref[...], preferred_element_type=jnp.float32)
    h = jnp.maximum(h + b2_ref[...], 0.0).astype(jnp.bfloat16)
    logits = jnp.dot(h, w3_ref[...], preferred_element_type=jnp.float32)
    logits = logits + b3_ref[...]
    m = jnp.max(logits, axis=1, keepdims=True)
    lse = jnp.log(jnp.sum(jnp.exp(logits - m), axis=1, keepdims=True)) + m
    o_ref[...] = logits - lse


BM = 512

_mlp = pl.pallas_call(
    _mlp_body,
    grid=(BC // BM,),
    in_specs=[
        pl.BlockSpec((BM, EMBP), lambda i: (i, 0)),
        pl.BlockSpec((EMBP, HID), lambda i: (0, 0)),
        pl.BlockSpec((1, HID), lambda i: (0, 0)),
        pl.BlockSpec((HID, HID), lambda i: (0, 0)),
        pl.BlockSpec((1, HID), lambda i: (0, 0)),
        pl.BlockSpec((HID, 2), lambda i: (0, 0)),
        pl.BlockSpec((1, 2), lambda i: (0, 0)),
    ],
    out_specs=pl.BlockSpec((BM, 2), lambda i: (i, 0)),
    out_shape=jax.ShapeDtypeStruct((BC, 2), jnp.float32),
    compiler_params=pltpu.CompilerParams(
        dimension_semantics=("arbitrary",)),
)


def kernel(x, emb, W1, b1, W2, b2, W3, b3):
    emb_p = _trans(emb.T, jnp.eye(EMB, EMBP, dtype=jnp.bfloat16))
    w1_p = jnp.pad(W1, ((0, EMBP - EMB), (0, 0)))
    w1b = w1_p.astype(jnp.bfloat16)
    w2b = W2.astype(jnp.bfloat16)
    w3b = W3.astype(jnp.bfloat16)
    b1r = b1.reshape(1, HID)
    b2r = b2.reshape(1, HID)
    b3r = b3.reshape(1, 2)
    xf = x.reshape(-1)
    pooled = [_sc_pool(lax.dynamic_slice_in_dim(xf, k * BC * L, BC * L),
                       emb_p)
              for k in range(NCHUNK)]
    outs = [_mlp(p.astype(jnp.bfloat16), w1b, b1r, w2b, b2r, w3b, b3r)
            for p in pooled]
    return jnp.concatenate(outs, axis=0)


# speedup vs baseline: 3.6490x; 1.1249x over previous
import functools

import jax
import jax.numpy as jnp
from jax import lax
from jax.experimental import pallas as pl
from jax.experimental.pallas import tpu as pltpu
from jax.experimental.pallas import tpu_sc as plsc

VOCAB = 100000
EMB = 300
HID = 4096
B = 4096
L = 200

NC = 2
NS = 16
NW = NC * NS
NCHUNK = 4
BC = B // NCHUNK
BPW = BC // NW
LA = 104
LB = L - LA

EMBP = 384
HALF = EMBP // 2
PACKW = 256

_NCHP = HALF // 16


def _sc_pool_body(x_hbm, emb_hbm, out_hbm, *refs):
    idxb = refs[0:2]
    bufs = refs[2:4]
    out_v = refs[4]
    gsem = refs[5:7]
    isem = refs[7:9]

    wid = lax.axis_index("s") * NC + lax.axis_index("c")
    base = wid * BPW
    zero = jnp.zeros((16,), jnp.float32)
    scale = jnp.full((16,), 1.0 / L, jnp.float32)

    def fire_idx(i, slot):
        pltpu.async_copy(x_hbm.at[pl.ds((base + i) * L, L)],
                         idxb[slot], isem[slot])

    def wait_idx(i, slot):
        pltpu.make_async_copy(x_hbm.at[pl.ds((base + i) * L, L)],
                              idxb[slot], isem[slot]).wait()

    def gather_a(slot):
        return pltpu.make_async_copy(
            emb_hbm.at[idxb[slot].at[pl.ds(0, LA)]], bufs[0], gsem[0])

    def gather_b(slot):
        return pltpu.make_async_copy(
            emb_hbm.at[idxb[slot].at[pl.ds(LA, LB)]], bufs[1], gsem[1])

    def reduce_rows(buf, n):
        @pl.loop(0, n, init_carry=(zero,) * (2 * _NCHP))
        def sums(r, carry):
            vals = []
            for c in range(_NCHP):
                w = buf[r, pl.ds(16 * c, 16)]
                lo, hi = plsc.unpack(plsc.bitcast(w, jnp.bfloat16),
                                     format=plsc.PackFormat.INTERLEAVED)
                vals.append(carry[2 * c] + lo)
                vals.append(carry[2 * c + 1] + hi)
            return tuple(vals)
        return sums

    fire_idx(0, 0)
    fire_idx(1, 1)
    wait_idx(0, 0)
    gather_a(0).start()

    @pl.loop(0, BPW // 2)
    def _(t):
        for j in range(2):
            i = t * 2 + j
            gather_a(j).wait()
            gather_b(j).start()
            sums_a = reduce_rows(bufs[0], LA)
            for c in range(_NCHP):
                out_v[i, pl.ds(16 * c, 16)] = sums_a[2 * c]
                out_v[i, pl.ds(HALF + 16 * c, 16)] = sums_a[2 * c + 1]
            gather_b(j).wait()

            @pl.when(i + 1 < BPW)
            def _():
                wait_idx(i + 1, 1 - j)
                gather_a(1 - j).start()

            @pl.when(i + 2 < BPW)
            def _():
                fire_idx(i + 2, j)

            sums_b = reduce_rows(bufs[1], LB)
            for c in range(_NCHP):
                out_v[i, pl.ds(16 * c, 16)] = (
                    out_v[i, pl.ds(16 * c, 16)] + sums_b[2 * c]) * scale
                out_v[i, pl.ds(HALF + 16 * c, 16)] = (
                    out_v[i, pl.ds(HALF + 16 * c, 16)]
                    + sums_b[2 * c + 1]) * scale

    pltpu.sync_copy(out_v, out_hbm.at[pl.ds(base, BPW)])


_sc_pool = functools.partial(
    pl.kernel,
    out_type=jax.ShapeDtypeStruct((BC, EMBP), jnp.float32),
    mesh=plsc.VectorSubcoreMesh(core_axis_name="c", subcore_axis_name="s"),
    scratch_types=(
        [pltpu.VMEM((L,), jnp.int32) for _ in range(2)]
        + [pltpu.VMEM((LA, PACKW), jnp.float32),
           pltpu.VMEM((LB, PACKW), jnp.float32)]
        + [pltpu.VMEM((BPW, EMBP), jnp.float32)]
        + [pltpu.SemaphoreType.DMA for _ in range(4)]
    ),
    compiler_params=pltpu.CompilerParams(needs_layout_passes=False),
)(_sc_pool_body)


def _trans_body(a_ref, i_ref, o_ref):
    res = lax.dot_general(
        a_ref[...].astype(jnp.bfloat16), i_ref[...],
        dimension_numbers=(((0,), (0,)), ((), ())),
        preferred_element_type=jnp.float32)
    bits = lax.bitcast_convert_type(res, jnp.uint32) + jnp.uint32(0x8000)
    lo = bits[:, :HALF] >> 16
    hi = bits[:, HALF:] & jnp.uint32(0xFFFF0000)
    o_ref[:, :HALF] = lax.bitcast_convert_type(lo | hi, jnp.float32)
    o_ref[:, HALF:] = jnp.zeros((o_ref.shape[0], PACKW - HALF), jnp.float32)


_TV = 2048

_trans = pl.pallas_call(
    _trans_body,
    grid=((VOCAB + _TV - 1) // _TV,),
    in_specs=[
        pl.BlockSpec((EMB, _TV), lambda i: (0, i)),
        pl.BlockSpec((EMB, EMBP), lambda i: (0, 0)),
    ],
    out_specs=pl.BlockSpec((_TV, PACKW), lambda i: (i, 0)),
    out_shape=jax.ShapeDtypeStruct((VOCAB, PACKW), jnp.float32),
    compiler_params=pltpu.CompilerParams(
        dimension_semantics=("arbitrary",)),
)


def _mlp_body(x_ref, w1_ref, b1_ref, w2_ref, b2_ref, w3_ref, b3_ref, o_ref):
    h = jnp.dot(x_ref[...], w1_ref[...], preferred_element_type=jnp.float32)
    h = jnp.maximum(h + b1_ref[...], 0.0).astype(jnp.bfloat16)
    h = jnp.dot(h, w2_ref[...], preferred_element_type=jnp.float32)
    h = jnp.maximum(h + b2_ref[...], 0.0).astype(jnp.bfloat16)
    logits = jnp.dot(h, w3_ref[...], preferred_element_type=jnp.float32)
    logits = logits + b3_ref[...]
    m = jnp.max(logits, axis=1, keepdims=True)
    lse = jnp.log(jnp.sum(jnp.exp(logits - m), axis=1, keepdims=True)) + m
    o_ref[...] = logits - lse


BM = 512

_mlp = pl.pallas_call(
    _mlp_body,
    grid=(BC // BM,),
    in_specs=[
        pl.BlockSpec((BM, EMBP), lambda i: (i, 0)),
        pl.BlockSpec((EMBP, HID), lambda i: (0, 0)),
        pl.BlockSpec((1, HID), lambda i: (0, 0)),
        pl.BlockSpec((HID, HID), lambda i: (0, 0)),
        pl.BlockSpec((1, HID), lambda i: (0, 0)),
        pl.BlockSpec((HID, 2), lambda i: (0, 0)),
        pl.BlockSpec((1, 2), lambda i: (0, 0)),
    ],
    out_specs=pl.BlockSpec((BM, 2), lambda i: (i, 0)),
    out_shape=jax.ShapeDtypeStruct((BC, 2), jnp.float32),
    compiler_params=pltpu.CompilerParams(
        dimension_semantics=("arbitrary",)),
)


def kernel(x, emb, W1, b1, W2, b2, W3, b3):
    emb_p = _trans(emb.T, jnp.eye(EMB, EMBP, dtype=jnp.bfloat16))
    w1_p = jnp.pad(W1, ((0, EMBP - EMB), (0, 0)))
    w1b = w1_p.astype(jnp.bfloat16)
    w2b = W2.astype(jnp.bfloat16)
    w3b = W3.astype(jnp.bfloat16)
    b1r = b1.reshape(1, HID)
    b2r = b2.reshape(1, HID)
    b3r = b3.reshape(1, 2)
    xf = x.reshape(-1)
    pooled = [_sc_pool(lax.dynamic_slice_in_dim(xf, k * BC * L, BC * L),
                       emb_p)
              for k in range(NCHUNK)]
    outs = [_mlp(p.astype(jnp.bfloat16), w1b, b1r, w2b, b2r, w3b, b3r)
            for p in pooled]
    return jnp.concatenate(outs, axis=0)


# speedup vs baseline: 4.4984x; 1.2328x over previous
import functools

import jax
import jax.numpy as jnp
from jax import lax
from jax.experimental import pallas as pl
from jax.experimental.pallas import tpu as pltpu
from jax.experimental.pallas import tpu_sc as plsc

VOCAB = 100000
EMB = 300
HID = 4096
B = 4096
L = 200

NC = 2
NS = 16
NW = NC * NS
NCHUNK = 4
BC = B // NCHUNK
BPW = BC // NW
LA = 104
LB = L - LA

EMBP = 384
HALF = EMBP // 2
PACKW = 256

_NCHP = HALF // 16


NU = 2 * BPW


def _sc_pool_body(x_hbm, emb_hbm, out_hbm, *refs):
    idxb = refs[0:2]
    bufs = refs[2:6]
    out_v = refs[6]
    gsem = refs[7:11]
    isem = refs[11:13]

    wid = lax.axis_index("s") * NC + lax.axis_index("c")
    base = wid * BPW
    zero = jnp.zeros((16,), jnp.float32)
    scale = jnp.full((16,), 1.0 / L, jnp.float32)

    def fire_idx(r, slot):
        pltpu.async_copy(x_hbm.at[pl.ds((base + r) * L, L)],
                         idxb[slot], isem[slot])

    def wait_idx(r, slot):
        pltpu.make_async_copy(x_hbm.at[pl.ds((base + r) * L, L)],
                              idxb[slot], isem[slot]).wait()

    def gather(islot, half, bslot):
        if half == 0:
            src = emb_hbm.at[idxb[islot].at[pl.ds(0, LA)]]
            dst = bufs[bslot]
        else:
            src = emb_hbm.at[idxb[islot].at[pl.ds(LA, LB)]]
            dst = bufs[bslot].at[pl.ds(0, LB)]
        return pltpu.make_async_copy(src, dst, gsem[bslot])

    def reduce_rows(buf, n):
        @pl.loop(0, n, init_carry=(zero,) * (2 * _NCHP))
        def sums(r, carry):
            vals = []
            for c in range(_NCHP):
                w = buf[r, pl.ds(16 * c, 16)]
                lo, hi = plsc.unpack(plsc.bitcast(w, jnp.bfloat16),
                                     format=plsc.PackFormat.INTERLEAVED)
                vals.append(carry[2 * c] + lo)
                vals.append(carry[2 * c + 1] + hi)
            return tuple(vals)
        return sums

    fire_idx(0, 0)
    fire_idx(1, 1)
    wait_idx(0, 0)
    gather(0, 0, 0).start()
    gather(0, 1, 1).start()

    @pl.loop(0, NU // 8)
    def _(t):
        for j in range(8):
            u = t * 8 + j
            r = u // 2
            rs = (j // 2) % 2
            gather(rs, j % 2, j % 4).wait()

            @pl.when(u + 2 < NU)
            def _():
                if j % 2 == 0:
                    wait_idx(r + 1, 1 - rs)
                gather(1 - rs, j % 2, (j + 2) % 4).start()

            if j % 2 == 1:
                @pl.when(r + 2 < BPW)
                def _():
                    fire_idx(r + 2, rs)

            if j % 2 == 0:
                sums = reduce_rows(bufs[j % 4], LA)
                for c in range(_NCHP):
                    out_v[r, pl.ds(16 * c, 16)] = sums[2 * c]
                    out_v[r, pl.ds(HALF + 16 * c, 16)] = sums[2 * c + 1]
            else:
                sums = reduce_rows(bufs[j % 4], LB)
                for c in range(_NCHP):
                    out_v[r, pl.ds(16 * c, 16)] = (
                        out_v[r, pl.ds(16 * c, 16)] + sums[2 * c]) * scale
                    out_v[r, pl.ds(HALF + 16 * c, 16)] = (
                        out_v[r, pl.ds(HALF + 16 * c, 16)]
                        + sums[2 * c + 1]) * scale

    pltpu.sync_copy(out_v, out_hbm.at[pl.ds(base, BPW)])


_sc_pool = functools.partial(
    pl.kernel,
    out_type=jax.ShapeDtypeStruct((BC, EMBP), jnp.float32),
    mesh=plsc.VectorSubcoreMesh(core_axis_name="c", subcore_axis_name="s"),
    scratch_types=(
        [pltpu.VMEM((L,), jnp.int32) for _ in range(2)]
        + [pltpu.VMEM((LA, PACKW), jnp.float32) for _ in range(4)]
        + [pltpu.VMEM((BPW, EMBP), jnp.float32)]
        + [pltpu.SemaphoreType.DMA for _ in range(6)]
    ),
    compiler_params=pltpu.CompilerParams(needs_layout_passes=False),
)(_sc_pool_body)


def _trans_body(a_ref, i_ref, o_ref):
    res = lax.dot_general(
        a_ref[...].astype(jnp.bfloat16), i_ref[...],
        dimension_numbers=(((0,), (0,)), ((), ())),
        preferred_element_type=jnp.float32)
    bits = lax.bitcast_convert_type(res, jnp.uint32) + jnp.uint32(0x8000)
    lo = bits[:, :HALF] >> 16
    hi = bits[:, HALF:] & jnp.uint32(0xFFFF0000)
    o_ref[:, :HALF] = lax.bitcast_convert_type(lo | hi, jnp.float32)
    o_ref[:, HALF:] = jnp.zeros((o_ref.shape[0], PACKW - HALF), jnp.float32)


_TV = 2048

_trans = pl.pallas_call(
    _trans_body,
    grid=((VOCAB + _TV - 1) // _TV,),
    in_specs=[
        pl.BlockSpec((EMB, _TV), lambda i: (0, i)),
        pl.BlockSpec((EMB, EMBP), lambda i: (0, 0)),
    ],
    out_specs=pl.BlockSpec((_TV, PACKW), lambda i: (i, 0)),
    out_shape=jax.ShapeDtypeStruct((VOCAB, PACKW), jnp.float32),
    compiler_params=pltpu.CompilerParams(
        dimension_semantics=("arbitrary",)),
)


def _mlp_body(x_ref, w1_ref, b1_ref, w2_ref, b2_ref, w3_ref, b3_ref, o_ref):
    h = jnp.dot(x_ref[...], w1_ref[...], preferred_element_type=jnp.float32)
    h = jnp.maximum(h + b1_ref[...], 0.0).astype(jnp.bfloat16)
    h = jnp.dot(h, w2_ref[...], preferred_element_type=jnp.float32)
    h = jnp.maximum(h + b2_ref[...], 0.0).astype(jnp.bfloat16)
    logits = jnp.dot(h, w3_ref[...], preferred_element_type=jnp.float32)
    logits = logits + b3_ref[...]
    m = jnp.max(logits, axis=1, keepdims=True)
    lse = jnp.log(jnp.sum(jnp.exp(logits - m), axis=1, keepdims=True)) + m
    o_ref[...] = logits - lse


BM = 512

_mlp = pl.pallas_call(
    _mlp_body,
    grid=(BC // BM,),
    in_specs=[
        pl.BlockSpec((BM, EMBP), lambda i: (i, 0)),
        pl.BlockSpec((EMBP, HID), lambda i: (0, 0)),
        pl.BlockSpec((1, HID), lambda i: (0, 0)),
        pl.BlockSpec((HID, HID), lambda i: (0, 0)),
        pl.BlockSpec((1, HID), lambda i: (0, 0)),
        pl.BlockSpec((HID, 2), lambda i: (0, 0)),
        pl.BlockSpec((1, 2), lambda i: (0, 0)),
    ],
    out_specs=pl.BlockSpec((BM, 2), lambda i: (i, 0)),
    out_shape=jax.ShapeDtypeStruct((BC, 2), jnp.float32),
    compiler_params=pltpu.CompilerParams(
        dimension_semantics=("arbitrary",)),
)


def kernel(x, emb, W1, b1, W2, b2, W3, b3):
    emb_p = _trans(emb.T, jnp.eye(EMB, EMBP, dtype=jnp.bfloat16))
    w1_p = jnp.pad(W1, ((0, EMBP - EMB), (0, 0)))
    w1b = w1_p.astype(jnp.bfloat16)
    w2b = W2.astype(jnp.bfloat16)
    w3b = W3.astype(jnp.bfloat16)
    b1r = b1.reshape(1, HID)
    b2r = b2.reshape(1, HID)
    b3r = b3.reshape(1, 2)
    xf = x.reshape(-1)
    pooled = [_sc_pool(lax.dynamic_slice_in_dim(xf, k * BC * L, BC * L),
                       emb_p)
              for k in range(NCHUNK)]
    outs = [_mlp(p.astype(jnp.bfloat16), w1b, b1r, w2b, b2r, w3b, b3r)
            for p in pooled]
    return jnp.concatenate(outs, axis=0)
